# baseline (speedup 1.0000x reference)
import functools

import jax
import jax.numpy as jnp
from jax import lax
from jax.experimental import pallas as pl
from jax.experimental.pallas import tpu as pltpu

EPS = 1e-5
PAD = 5


def _branches_kernel(x_ref, w_ref, ycat_ref, stats_ref, s_ref,
                     *, D, H, W, C):
    j = pl.program_id(1)
    d0 = 4 * j
    Dp = D + 2 * PAD
    hw2 = 4 * H * W

    def build(b, dil, p):
        pr = jnp.minimum(p, Dp - 1)
        pw = jnp.minimum(p, Dp)
        for kw in range(3):
            c0 = PAD + (kw - 1) * dil
            s_ref[b, pl.ds(pw, 1), :, :, kw * C:(kw + 1) * C] = (
                x_ref[0, pl.ds(pr, 1), :, pl.ds(c0, W), :])

    @pl.when(j == 0)
    def _():
        for b, dil in enumerate((1, 3, 5)):
            for p in range(PAD - dil, PAD + dil + 4):
                build(b, dil, p)

    for b, dil in enumerate((1, 3, 5)):
        parts = []
        for kd in range(3):
            dp = d0 + PAD + (kd - 1) * dil
            for kh in range(3):
                r0 = PAD + (kh - 1) * dil
                parts.append(s_ref[b, pl.ds(dp, 4), pl.ds(r0, H), :, :]
                             .reshape(hw2, 3 * C))
        a = jnp.concatenate(parts, axis=1)
        acc = jnp.dot(a, w_ref[b], preferred_element_type=jnp.float32)
        ycat_ref[0, :, :, :, b * C:(b + 1) * C] = (
            acc.reshape(4, H, W, C).astype(jnp.bfloat16))
        stats_ref[0, 0:1, b * C:(b + 1) * C] = jnp.sum(acc, 0, keepdims=True)
        stats_ref[0, 1:2, b * C:(b + 1) * C] = jnp.sum(acc * acc, 0,
                                                       keepdims=True)

    for b, dil in enumerate((1, 3, 5)):
        for q in range(9, 13):
            build(b, dil, d0 + q + dil)


def _fuse_kernel(y_ref, w_ref, sc_ref, sh_ref, yf_ref, stats_ref,
                 s_ref, pn_ref, *, D, H, W, C, CO):
    j = pl.program_id(1)
    d0 = 4 * j
    sc = sc_ref[...].reshape(1, 1, C)
    sh = sh_ref[...].reshape(1, 1, C)
    hw2 = 4 * H * W

    def build(pd):
        di = pd - 1
        dc = jnp.clip(di, 0, D - 1)
        raw = y_ref[0, pl.ds(dc, 1), :, :, :].reshape(H, W, C)
        norm = jnp.maximum(raw.astype(jnp.float32) * sc + sh, 0.0)
        inb = jnp.logical_and(di >= 0, di < D)
        pn_ref[...] = jnp.zeros((H + 2, W + 2, C), jnp.bfloat16)
        pn_ref[1:H + 1, 1:W + 1, :] = jnp.where(inb, norm, 0.0).astype(
            jnp.bfloat16)
        for kw in range(3):
            s_ref[pl.ds(pd, 1), :, :, kw * C:(kw + 1) * C] = (
                pn_ref[:, pl.ds(kw, W), :].reshape(1, H + 2, W, C))

    @pl.when(j == 0)
    def _():
        for pd in range(6):
            build(pd)

    parts = []
    for kd in range(3):
        dp = d0 + kd
        for kh in range(3):
            parts.append(s_ref[pl.ds(dp, 4), pl.ds(kh, H), :, :]
                         .reshape(hw2, 3 * C))
    a = jnp.concatenate(parts, axis=1)
    acc = jnp.dot(a, w_ref[...], preferred_element_type=jnp.float32)
    yf_ref[0] = acc.reshape(4, H, W, CO).astype(jnp.bfloat16)
    stats_ref[0, 0:1, :] = jnp.sum(acc, 0, keepdims=True)
    stats_ref[0, 1:2, :] = jnp.sum(acc * acc, 0, keepdims=True)
    for q in range(6, 10):
        build(d0 + q)


def _bn_relu_kernel2(y_ref, sc_ref, sh_ref, o_ref):
    y = y_ref[...].astype(jnp.float32)
    o_ref[...] = jnp.maximum(y * sc_ref[...].reshape(1, 1, 1, 1, -1)
                             + sh_ref[...].reshape(1, 1, 1, 1, -1), 0.0)


def _fold_stats(stats, gamma, beta, m):
    s = jnp.sum(stats[:, 0, :], axis=0)
    ss = jnp.sum(stats[:, 1, :], axis=0)
    mean = s / m
    var = ss / m - mean * mean
    scale = gamma.reshape(-1) * lax.rsqrt(var + EPS)
    shift = beta.reshape(-1) - mean * scale
    return scale.reshape(1, -1), shift.reshape(1, -1)


@jax.jit
def kernel(x, b1_w, b1_b, b1_g, b1_bt, b3_w, b3_b, b3_g, b3_bt,
           b5_w, b5_b, b5_g, b5_bt, fuse_w, fuse_b, fuse_g, fuse_bt):
    N, C, D, H, W = x.shape
    M = N * D * H * W
    C3 = 3 * C
    D4 = D // 4
    G = N * D

    xt = jnp.transpose(x, (0, 2, 3, 4, 1))
    xp = jnp.pad(xt, ((0, 0), (PAD, PAD), (PAD, PAD), (PAD, PAD), (0, 0)))
    xp = xp.astype(jnp.bfloat16)
    wb = jnp.stack([b1_w.reshape(27 * C, C), b3_w.reshape(27 * C, C),
                    b5_w.reshape(27 * C, C)]).astype(jnp.bfloat16)
    wf = fuse_w.reshape(27 * C3, C).astype(jnp.bfloat16)
    Dp, Hp, Wp = D + 2 * PAD, H + 2 * PAD, W + 2 * PAD

    ycat, st1 = pl.pallas_call(
        functools.partial(_branches_kernel, D=D, H=H, W=W, C=C),
        out_shape=(jax.ShapeDtypeStruct((N, D, H, W, C3), jnp.bfloat16),
                   jax.ShapeDtypeStruct((N * D4, 8, C3), jnp.float32)),
        grid=(N, D4),
        in_specs=[
            pl.BlockSpec((1, Dp, Hp, Wp, C), lambda n, j: (n, 0, 0, 0, 0)),
            pl.BlockSpec((3, 27 * C, C), lambda n, j: (0, 0, 0)),
        ],
        out_specs=(
            pl.BlockSpec((1, 4, H, W, C3), lambda n, j: (n, j, 0, 0, 0)),
            pl.BlockSpec((1, 8, C3), lambda n, j: (n * D4 + j, 0, 0)),
        ),
        scratch_shapes=[pltpu.VMEM((3, Dp + 1, Hp, W, C3), jnp.bfloat16)],
        compiler_params=pltpu.CompilerParams(
            dimension_semantics=("arbitrary", "arbitrary")),
        cost_estimate=pl.CostEstimate(
            flops=2 * M * 3 * 27 * C * C, transcendentals=0,
            bytes_accessed=2 * (N * Dp * Hp * Wp * C + M * C3)),
    )(xp, wb)
    sc1, sh1 = _fold_stats(st1, jnp.concatenate(
        [b1_g.reshape(-1), b3_g.reshape(-1), b5_g.reshape(-1)]),
        jnp.concatenate(
        [b1_bt.reshape(-1), b3_bt.reshape(-1), b5_bt.reshape(-1)]), M)

    yf, st2 = pl.pallas_call(
        functools.partial(_fuse_kernel, D=D, H=H, W=W, C=C3, CO=C),
        out_shape=(jax.ShapeDtypeStruct((N, D, H, W, C), jnp.bfloat16),
                   jax.ShapeDtypeStruct((N * D4, 8, C), jnp.float32)),
        grid=(N, D4),
        in_specs=[
            pl.BlockSpec((1, D, H, W, C3), lambda n, j: (n, 0, 0, 0, 0)),
            pl.BlockSpec((27 * C3, C), lambda n, j: (0, 0)),
            pl.BlockSpec((1, C3), lambda n, j: (0, 0)),
            pl.BlockSpec((1, C3), lambda n, j: (0, 0)),
        ],
        out_specs=(
            pl.BlockSpec((1, 4, H, W, C), lambda n, j: (n, j, 0, 0, 0)),
            pl.BlockSpec((1, 8, C), lambda n, j: (n * D4 + j, 0, 0)),
        ),
        scratch_shapes=[
            pltpu.VMEM((D + 7, H + 2, W, 3 * C3), jnp.bfloat16),
            pltpu.VMEM((H + 2, W + 2, C3), jnp.bfloat16),
        ],
        compiler_params=pltpu.CompilerParams(
            dimension_semantics=("arbitrary", "arbitrary")),
        cost_estimate=pl.CostEstimate(
            flops=2 * M * 27 * C3 * C, transcendentals=0,
            bytes_accessed=2 * (M * C3 + M * C)),
    )(ycat, wf, sc1, sh1)
    sc2, sh2 = _fold_stats(st2, fuse_g, fuse_bt, M)

    out = pl.pallas_call(
        _bn_relu_kernel2,
        out_shape=jax.ShapeDtypeStruct((N, D, H, W, C), jnp.float32),
        grid=(G,),
        in_specs=[
            pl.BlockSpec((1, 1, H, W, C), lambda i: (i // D, i % D, 0, 0, 0)),
            pl.BlockSpec((1, C), lambda i: (0, 0)),
            pl.BlockSpec((1, C), lambda i: (0, 0)),
        ],
        out_specs=pl.BlockSpec((1, 1, H, W, C),
                               lambda i: (i // D, i % D, 0, 0, 0)),
        compiler_params=pltpu.CompilerParams(
            dimension_semantics=("arbitrary",)),
    )(yf, sc2, sh2)
    return jnp.transpose(out, (0, 4, 1, 2, 3))


# speedup vs baseline: 1.1344x; 1.1344x over previous
import functools

import jax
import jax.numpy as jnp
from jax import lax
from jax.experimental import pallas as pl
from jax.experimental.pallas import tpu as pltpu

EPS = 1e-5
PAD = 5


def _branches_kernel(x_ref, w_ref, ycat_ref, stats_ref, s_ref,
                     *, D, H, W, C):
    j = pl.program_id(1)
    d0 = 4 * j
    Dp = D + 2 * PAD
    hw2 = 4 * H * W

    def build(b, dil, p):
        pr = jnp.minimum(p, Dp - 1)
        pw = jnp.minimum(p, Dp)
        for kw in range(3):
            c0 = PAD + (kw - 1) * dil
            s_ref[b, pl.ds(pw, 1), :, :, kw * C:(kw + 1) * C] = (
                x_ref[0, pl.ds(pr, 1), :, pl.ds(c0, W), :])

    @pl.when(j == 0)
    def _():
        for b, dil in enumerate((1, 3, 5)):
            for p in range(PAD - dil, PAD + dil + 4):
                build(b, dil, p)

    for b, dil in enumerate((1, 3, 5)):
        for q in range(9, 13):
            build(b, dil, d0 + q + dil)

    for b, dil in enumerate((1, 3, 5)):
        parts = []
        for kd in range(3):
            dp = d0 + PAD + (kd - 1) * dil
            for kh in range(3):
                r0 = PAD + (kh - 1) * dil
                parts.append(s_ref[b, pl.ds(dp, 4), pl.ds(r0, H), :, :]
                             .reshape(hw2, 3 * C))
        a = jnp.concatenate(parts, axis=1)
        acc = jnp.dot(a, w_ref[b], preferred_element_type=jnp.float32)
        ycat_ref[0, :, :, :, b * C:(b + 1) * C] = (
            acc.reshape(4, H, W, C).astype(jnp.bfloat16))
        stats_ref[0, 0:1, b * C:(b + 1) * C] = jnp.sum(acc, 0, keepdims=True)
        stats_ref[0, 1:2, b * C:(b + 1) * C] = jnp.sum(acc * acc, 0,
                                                       keepdims=True)


def _fuse_kernel(y_ref, w_ref, st_ref, g_ref, bt_ref, yf_ref, stats_ref,
                 s_ref, pn_ref, *, D, H, W, C, CO, M):
    j = pl.program_id(1)
    d0 = 4 * j
    s = jnp.sum(st_ref[:, 0, :], axis=0, keepdims=True)
    ss = jnp.sum(st_ref[:, 1, :], axis=0, keepdims=True)
    mean = s / M
    var = ss / M - mean * mean
    sc = (g_ref[...] * lax.rsqrt(var + EPS)).reshape(1, 1, C)
    sh = bt_ref[...].reshape(1, 1, C) - mean.reshape(1, 1, C) * sc
    hw2 = 4 * H * W

    def build(pd):
        di = pd - 1
        dc = jnp.clip(di, 0, D - 1)
        raw = y_ref[0, pl.ds(dc, 1), :, :, :].reshape(H, W, C)
        norm = jnp.maximum(raw.astype(jnp.float32) * sc + sh, 0.0)
        inb = jnp.logical_and(di >= 0, di < D)
        pn_ref[...] = jnp.zeros((H + 2, W + 2, C), jnp.bfloat16)
        pn_ref[1:H + 1, 1:W + 1, :] = jnp.where(inb, norm, 0.0).astype(
            jnp.bfloat16)
        for kw in range(3):
            s_ref[pl.ds(pd, 1), :, :, kw * C:(kw + 1) * C] = (
                pn_ref[:, pl.ds(kw, W), :].reshape(1, H + 2, W, C))

    @pl.when(j == 0)
    def _():
        for pd in range(6):
            build(pd)

    for q in range(6, 10):
        build(d0 + q)

    parts = []
    for kd in range(3):
        dp = d0 + kd
        for kh in range(3):
            parts.append(s_ref[pl.ds(dp, 4), pl.ds(kh, H), :, :]
                         .reshape(hw2, 3 * C))
    a = jnp.concatenate(parts, axis=1)
    acc = jnp.dot(a, w_ref[...], preferred_element_type=jnp.float32)
    yf_ref[0] = acc.reshape(4, H, W, CO).astype(jnp.bfloat16)
    stats_ref[0, 0:1, :] = jnp.sum(acc, 0, keepdims=True)
    stats_ref[0, 1:2, :] = jnp.sum(acc * acc, 0, keepdims=True)


def _bn_relu_kernel2(y_ref, st_ref, g_ref, bt_ref, o_ref, *, M):
    C = y_ref.shape[-1]
    s = jnp.sum(st_ref[:, 0, :], axis=0, keepdims=True)
    ss = jnp.sum(st_ref[:, 1, :], axis=0, keepdims=True)
    mean = s / M
    var = ss / M - mean * mean
    sc = (g_ref[...] * lax.rsqrt(var + EPS)).reshape(1, 1, 1, 1, C)
    sh = (bt_ref[...] - mean * g_ref[...] * lax.rsqrt(var + EPS)
          ).reshape(1, 1, 1, 1, C)
    y = y_ref[...].astype(jnp.float32)
    o_ref[...] = jnp.maximum(y * sc + sh, 0.0)


def _fold_stats(stats, gamma, beta, m):
    s = jnp.sum(stats[:, 0, :], axis=0)
    ss = jnp.sum(stats[:, 1, :], axis=0)
    mean = s / m
    var = ss / m - mean * mean
    scale = gamma.reshape(-1) * lax.rsqrt(var + EPS)
    shift = beta.reshape(-1) - mean * scale
    return scale.reshape(1, -1), shift.reshape(1, -1)


@jax.jit
def kernel(x, b1_w, b1_b, b1_g, b1_bt, b3_w, b3_b, b3_g, b3_bt,
           b5_w, b5_b, b5_g, b5_bt, fuse_w, fuse_b, fuse_g, fuse_bt):
    N, C, D, H, W = x.shape
    M = N * D * H * W
    C3 = 3 * C
    D4 = D // 4
    G = N * D

    xt = jnp.transpose(x, (0, 2, 3, 4, 1))
    xp = jnp.pad(xt, ((0, 0), (PAD, PAD), (PAD, PAD), (PAD, PAD), (0, 0)))
    xp = xp.astype(jnp.bfloat16)
    wb = jnp.stack([b1_w.reshape(27 * C, C), b3_w.reshape(27 * C, C),
                    b5_w.reshape(27 * C, C)]).astype(jnp.bfloat16)
    wf = fuse_w.reshape(27 * C3, C).astype(jnp.bfloat16)
    Dp, Hp, Wp = D + 2 * PAD, H + 2 * PAD, W + 2 * PAD

    ycat, st1 = pl.pallas_call(
        functools.partial(_branches_kernel, D=D, H=H, W=W, C=C),
        out_shape=(jax.ShapeDtypeStruct((N, D, H, W, C3), jnp.bfloat16),
                   jax.ShapeDtypeStruct((N * D4, 8, C3), jnp.float32)),
        grid=(N, D4),
        in_specs=[
            pl.BlockSpec((1, Dp, Hp, Wp, C), lambda n, j: (n, 0, 0, 0, 0)),
            pl.BlockSpec((3, 27 * C, C), lambda n, j: (0, 0, 0)),
        ],
        out_specs=(
            pl.BlockSpec((1, 4, H, W, C3), lambda n, j: (n, j, 0, 0, 0)),
            pl.BlockSpec((1, 8, C3), lambda n, j: (n * D4 + j, 0, 0)),
        ),
        scratch_shapes=[pltpu.VMEM((3, Dp + 1, Hp, W, C3), jnp.bfloat16)],
        compiler_params=pltpu.CompilerParams(
            dimension_semantics=("arbitrary", "arbitrary")),
        cost_estimate=pl.CostEstimate(
            flops=2 * M * 3 * 27 * C * C, transcendentals=0,
            bytes_accessed=2 * (N * Dp * Hp * Wp * C + M * C3)),
    )(xp, wb)
    g1 = jnp.concatenate([b1_g.reshape(-1), b3_g.reshape(-1),
                          b5_g.reshape(-1)]).reshape(1, C3)
    bt1 = jnp.concatenate([b1_bt.reshape(-1), b3_bt.reshape(-1),
                           b5_bt.reshape(-1)]).reshape(1, C3)

    yf, st2 = pl.pallas_call(
        functools.partial(_fuse_kernel, D=D, H=H, W=W, C=C3, CO=C, M=M),
        out_shape=(jax.ShapeDtypeStruct((N, D, H, W, C), jnp.bfloat16),
                   jax.ShapeDtypeStruct((N * D4, 8, C), jnp.float32)),
        grid=(N, D4),
        in_specs=[
            pl.BlockSpec((1, D, H, W, C3), lambda n, j: (n, 0, 0, 0, 0)),
            pl.BlockSpec((27 * C3, C), lambda n, j: (0, 0)),
            pl.BlockSpec((N * D4, 8, C3), lambda n, j: (0, 0, 0)),
            pl.BlockSpec((1, C3), lambda n, j: (0, 0)),
            pl.BlockSpec((1, C3), lambda n, j: (0, 0)),
        ],
        out_specs=(
            pl.BlockSpec((1, 4, H, W, C), lambda n, j: (n, j, 0, 0, 0)),
            pl.BlockSpec((1, 8, C), lambda n, j: (n * D4 + j, 0, 0)),
        ),
        scratch_shapes=[
            pltpu.VMEM((D + 7, H + 2, W, 3 * C3), jnp.bfloat16),
            pltpu.VMEM((H + 2, W + 2, C3), jnp.bfloat16),
        ],
        compiler_params=pltpu.CompilerParams(
            dimension_semantics=("arbitrary", "arbitrary")),
        cost_estimate=pl.CostEstimate(
            flops=2 * M * 27 * C3 * C, transcendentals=0,
            bytes_accessed=2 * (M * C3 + M * C)),
    )(ycat, wf, st1, g1, bt1)

    out = pl.pallas_call(
        functools.partial(_bn_relu_kernel2, M=M),
        out_shape=jax.ShapeDtypeStruct((N, D, H, W, C), jnp.float32),
        grid=(N, D4),
        in_specs=[
            pl.BlockSpec((1, 4, H, W, C), lambda n, j: (n, j, 0, 0, 0)),
            pl.BlockSpec((N * D4, 8, C), lambda n, j: (0, 0, 0)),
            pl.BlockSpec((1, C), lambda n, j: (0, 0)),
            pl.BlockSpec((1, C), lambda n, j: (0, 0)),
        ],
        out_specs=pl.BlockSpec((1, 4, H, W, C),
                               lambda n, j: (n, j, 0, 0, 0)),
        compiler_params=pltpu.CompilerParams(
            dimension_semantics=("arbitrary", "arbitrary")),
    )(yf, st2, fuse_g.reshape(1, C), fuse_bt.reshape(1, C))
    return jnp.transpose(out, (0, 4, 1, 2, 3))
